# Initial kernel scaffold; baseline (speedup 1.0000x reference)
import jax
import jax.numpy as jnp
from jax.experimental import pallas as pl


def kernel(x, batch, W1, b1, W2, b2):
    raise NotImplementedError("write your pallas kernel here")


# speedup vs baseline: 6.1346x; 6.1346x over previous
import jax
import jax.numpy as jnp
from jax import lax
from jax.experimental import pallas as pl
from jax.experimental.pallas import tpu as pltpu
from jax.experimental.pallas import tpu_sc as plsc

N = 320000
D = 128
S = 10000
H = 512
O = 128

NW = 32
SPS = 320
SPAD = NW * SPS
CHUNK = 256
NV = D // 16


CPAD = ((SPS + 16 + 15) // 16) * 16


def _pool_body(x_hbm, batch_hbm, starts_hbm,
               add_hbm, max_hbm, cnt_hbm,
               starts_v, bbuf, xbuf, sums, maxs, counts):
    cid = lax.axis_index("c")
    sid = lax.axis_index("s")
    wid = sid * 2 + cid
    lo = wid * SPS

    pltpu.sync_copy(starts_hbm, starts_v)
    rs8 = starts_v[pl.ds(wid, 16)][0]
    nxt = starts_v[pl.ds(wid + 1, 16)][0]
    re_eff = jnp.minimum(nxt + 8, N)

    zf = jnp.zeros((16,), jnp.float32)

    def zero_seg(i, _):
        for j in range(NV):
            sums[pl.ds(i * D + 16 * j, 16)] = zf
            maxs[pl.ds(i * D + 16 * j, 16)] = zf
        return 0

    lax.fori_loop(0, SPS, zero_seg, 0)

    def zero_cnt(i, _):
        counts[pl.ds(i * 16, 16)] = zf
        return 0

    lax.fori_loop(0, CPAD // 16, zero_cnt, 0)

    nch = (re_eff - rs8 + CHUNK - 1) // CHUNK

    def flush(c2):
        prev2, cnt2, ss2, ms2 = c2
        for j in range(NV):
            sums[pl.ds(prev2 * D + 16 * j, 16)] = ss2[j]
            maxs[pl.ds(prev2 * D + 16 * j, 16)] = ms2[j]
        counts[pl.ds(prev2, 16)] = jnp.full((16,), cnt2, jnp.float32)

    def chunk_body(ci, carry):
        lstart = rs8 + ci * CHUNK
        astart = jnp.minimum(lstart, N - CHUNK)
        astart = pl.multiple_of(astart, 8)
        delta = lstart - astart
        pltpu.sync_copy(batch_hbm.at[pl.ds(astart, CHUNK)], bbuf)
        xoff = pl.multiple_of(astart * D, 8)
        pltpu.sync_copy(x_hbm.at[pl.ds(xoff, CHUNK * D)], xbuf)

        def group_body(g, c):
            bvec = bbuf[pl.ds(16 * g, 16)]
            for l in range(16):
                prev, cnt, ss, ms = c
                r = 16 * g + l
                rel = bvec[l] - lo
                row_g = astart + r
                valid = ((r >= delta) & (row_g < re_eff)
                         & (rel >= 0) & (rel < SPS))
                vs = tuple(xbuf[pl.ds(r * D + 16 * j, 16)]
                           for j in range(NV))
                changed = valid & (rel != prev)

                @pl.when(changed & (cnt > 0.0))
                def _(prev=prev, cnt=cnt, ss=ss, ms=ms):
                    flush((prev, cnt, ss, ms))

                acc = valid & jnp.logical_not(changed)
                ss2 = tuple(
                    jnp.where(changed, vs[j],
                              jnp.where(acc, ss[j] + vs[j], ss[j]))
                    for j in range(NV))
                ms2 = tuple(
                    jnp.where(changed, vs[j],
                              jnp.where(acc, jnp.maximum(ms[j], vs[j]),
                                        ms[j]))
                    for j in range(NV))
                cnt2 = jnp.where(changed, 1.0,
                                 jnp.where(acc, cnt + 1.0, cnt))
                prev2 = jnp.where(changed, rel, prev)
                c = (prev2, cnt2, ss2, ms2)
            return c

        return lax.fori_loop(0, CHUNK // 16, group_body, carry)

    zvs = tuple(zf for _ in range(NV))
    init = (jnp.int32(-1), jnp.float32(0.0), zvs, zvs)
    final = lax.fori_loop(0, nch, chunk_body, init)

    @pl.when(final[1] > 0.0)
    def _():
        flush(final)

    loD = pl.multiple_of(lo * D, 8)
    lo8 = pl.multiple_of(lo, 8)
    pltpu.sync_copy(sums, add_hbm.at[pl.ds(loD, SPS * D)])
    pltpu.sync_copy(maxs, max_hbm.at[pl.ds(loD, SPS * D)])
    pltpu.sync_copy(counts.at[pl.ds(0, SPS)], cnt_hbm.at[pl.ds(lo8, SPS)])


def _mlp_body(cnt_ref, add_ref, max_ref, w1m, w1a, w1x, b1_ref,
              w2_ref, b2_ref, out_ref):
    inv = 1.0 / jnp.maximum(cnt_ref[...], 1.0)
    mean = add_ref[...] * inv
    h = (jnp.dot(mean, w1m[...], preferred_element_type=jnp.float32)
         + jnp.dot(add_ref[...], w1a[...], preferred_element_type=jnp.float32)
         + jnp.dot(max_ref[...], w1x[...], preferred_element_type=jnp.float32)
         + b1_ref[...])
    h = jnp.maximum(h, 0.0)
    out_ref[...] = (jnp.dot(h, w2_ref[...], preferred_element_type=jnp.float32)
                    + b2_ref[...])


BM = SPAD // 4


def kernel(x, batch, W1, b1, W2, b2):
    seg_bounds = jnp.arange(33, dtype=jnp.int32) * SPS
    row_start = jnp.searchsorted(batch, seg_bounds).astype(jnp.int32)
    rs8 = (row_start // 8) * 8
    starts = jnp.concatenate(
        [rs8, jnp.full((15,), N, dtype=jnp.int32)])

    mesh = plsc.VectorSubcoreMesh(core_axis_name="c", subcore_axis_name="s")
    pool = pl.kernel(
        _pool_body,
        out_type=[jax.ShapeDtypeStruct((SPAD * D,), jnp.float32),
                  jax.ShapeDtypeStruct((SPAD * D,), jnp.float32),
                  jax.ShapeDtypeStruct((SPAD,), jnp.float32)],
        mesh=mesh,
        scratch_types=[
            pltpu.VMEM((48,), jnp.int32),
            pltpu.VMEM((CHUNK,), jnp.int32),
            pltpu.VMEM((CHUNK * D,), jnp.float32),
            pltpu.VMEM((SPS * D,), jnp.float32),
            pltpu.VMEM((SPS * D,), jnp.float32),
            pltpu.VMEM((CPAD,), jnp.float32),
        ],
    )
    add_p, max_p, cnt_p = pool(x.reshape(N * D), batch, starts)
    add_p = add_p.reshape(SPAD, D)
    max_p = max_p.reshape(SPAD, D)
    cnt_p = cnt_p.reshape(SPAD, 1)

    out = pl.pallas_call(
        _mlp_body,
        grid=(SPAD // BM,),
        in_specs=[
            pl.BlockSpec((BM, 1), lambda i: (i, 0)),
            pl.BlockSpec((BM, D), lambda i: (i, 0)),
            pl.BlockSpec((BM, D), lambda i: (i, 0)),
            pl.BlockSpec((D, H), lambda i: (0, 0)),
            pl.BlockSpec((D, H), lambda i: (0, 0)),
            pl.BlockSpec((D, H), lambda i: (0, 0)),
            pl.BlockSpec((1, H), lambda i: (0, 0)),
            pl.BlockSpec((H, O), lambda i: (0, 0)),
            pl.BlockSpec((1, O), lambda i: (0, 0)),
        ],
        out_specs=pl.BlockSpec((BM, O), lambda i: (i, 0)),
        out_shape=jax.ShapeDtypeStruct((SPAD, O), jnp.float32),
    )(cnt_p, add_p, max_p,
      W1[0:D], W1[D:2 * D], W1[2 * D:3 * D],
      b1.reshape(1, H), W2, b2.reshape(1, O))
    return out[:S]


# speedup vs baseline: 8.2307x; 1.3417x over previous
import jax
import jax.numpy as jnp
from jax import lax
from jax.experimental import pallas as pl
from jax.experimental.pallas import tpu as pltpu
from jax.experimental.pallas import tpu_sc as plsc

N = 320000
D = 128
S = 10000
H = 512
O = 128

NW = 32
SPS = 320
SPAD = NW * SPS
CHUNK = 160
NV = D // 16
CPAD = SPS + 16
NEG = -3.0e38


def _pool_body(x_hbm, batch_hbm, starts_hbm,
               add_hbm, max_hbm, cnt_hbm,
               starts_v, bbuf0, bbuf1, xbuf0, xbuf1,
               sums, maxs, counts, semx0, semb0, semx1, semb1):
    cid = lax.axis_index("c")
    sid = lax.axis_index("s")
    wid = sid * 2 + cid
    lo = wid * SPS

    pltpu.sync_copy(starts_hbm, starts_v)
    rs8 = starts_v[pl.ds(wid, 16)][0]
    nxt = starts_v[pl.ds(wid + 1, 16)][0]
    re_eff = jnp.minimum(nxt + 8, N)

    zf = jnp.zeros((16,), jnp.float32)

    def zero_seg(i, _):
        for j in range(NV):
            sums[pl.ds(i * D + 16 * j, 16)] = zf
            maxs[pl.ds(i * D + 16 * j, 16)] = zf
        return 0

    lax.fori_loop(0, SPS, zero_seg, 0)

    def zero_cnt(i, _):
        counts[pl.ds(i * 16, 16)] = zf
        return 0

    lax.fori_loop(0, CPAD // 16, zero_cnt, 0)

    nch = (re_eff - rs8 + CHUNK - 1) // CHUNK
    nf_raw = (nxt - rs8) // CHUNK
    nf = jnp.maximum(1 + 2 * ((nf_raw - 1) // 2), 1)
    npairs = (nf - 1) // 2

    def flush(prev, cnt, ms):
        moff = prev * D
        for j in range(NV):
            maxs[pl.ds(moff + 16 * j, 16)] = ms[j]
        counts[pl.ds(prev, 16)] = jnp.full((16,), cnt, jnp.float32)

    def fast_rows(xb, bb, carry):
        def group_body(g, c):
            bvec = bb[pl.ds(16 * g, 16)]
            for l in range(16):
                prev, cnt, ms = c
                r = 16 * g + l
                rel = bvec[l] - lo
                off = rel * D
                vs = tuple(xb[pl.ds(r * D + 16 * j, 16)]
                           for j in range(NV))
                for j in range(NV):
                    plsc.addupdate(sums.at[pl.ds(off + 16 * j, 16)],
                                   vs[j])
                changed = rel != prev

                @pl.when(changed & (cnt > 0.0))
                def _(prev=prev, cnt=cnt, ms=ms):
                    flush(prev, cnt, ms)

                ms2 = tuple(jnp.where(changed, vs[j],
                                      jnp.maximum(ms[j], vs[j]))
                            for j in range(NV))
                cnt2 = jnp.where(changed, 1.0, cnt + 1.0)
                prev2 = jnp.where(changed, rel, prev)
                c = (prev2, cnt2, ms2)
            return c

        return lax.fori_loop(0, CHUNK // 16, group_body, carry)

    def slow_chunk(ci, carry):
        lstart = rs8 + ci * CHUNK
        astart = pl.multiple_of(jnp.minimum(lstart, N - CHUNK), 8)
        delta = lstart - astart
        pltpu.sync_copy(batch_hbm.at[pl.ds(astart, CHUNK)], bbuf0)
        xoff = pl.multiple_of(astart * D, 8)
        pltpu.sync_copy(x_hbm.at[pl.ds(xoff, CHUNK * D)], xbuf0)

        def group_body(g, c):
            bvec = bbuf0[pl.ds(16 * g, 16)]
            for l in range(16):
                prev, cnt, ms = c
                r = 16 * g + l
                rel = bvec[l] - lo
                row_g = astart + r
                valid = ((r >= delta) & (row_g < re_eff)
                         & (rel >= 0) & (rel < SPS))
                relc = jnp.where(valid, rel, 0)
                off = relc * D
                vs = tuple(xbuf0[pl.ds(r * D + 16 * j, 16)]
                           for j in range(NV))
                vz = tuple(jnp.where(valid, vs[j], 0.0)
                           for j in range(NV))
                for j in range(NV):
                    plsc.addupdate(sums.at[pl.ds(off + 16 * j, 16)],
                                   vz[j])
                changed = valid & (rel != prev)

                @pl.when(changed & (cnt > 0.0))
                def _(prev=prev, cnt=cnt, ms=ms):
                    flush(prev, cnt, ms)

                ms2 = tuple(
                    jnp.where(changed, vs[j],
                              jnp.maximum(ms[j],
                                          jnp.where(valid, vs[j], NEG)))
                    for j in range(NV))
                cnt2 = jnp.where(changed, 1.0,
                                 jnp.where(valid, cnt + 1.0, cnt))
                prev2 = jnp.where(changed, rel, prev)
                c = (prev2, cnt2, ms2)
            return c

        return lax.fori_loop(0, CHUNK // 16, group_body, carry)

    def start_dma(ci, xb, bb, sx, sb):
        a8 = pl.multiple_of(rs8 + ci * CHUNK, 8)
        pltpu.async_copy(batch_hbm.at[pl.ds(a8, CHUNK)], bb, sb)
        xo = pl.multiple_of(a8 * D, 8)
        pltpu.async_copy(x_hbm.at[pl.ds(xo, CHUNK * D)], xb, sx)

    def wait_dma(ci, xb, bb, sx, sb):
        a8 = pl.multiple_of(rs8 + ci * CHUNK, 8)
        pltpu.make_async_copy(batch_hbm.at[pl.ds(a8, CHUNK)], bb,
                              sb).wait()
        xo = pl.multiple_of(a8 * D, 8)
        pltpu.make_async_copy(x_hbm.at[pl.ds(xo, CHUNK * D)], xb,
                              sx).wait()

    zvs = tuple(zf for _ in range(NV))
    init = (jnp.int32(-1), jnp.float32(0.0), zvs)

    carry = lax.fori_loop(0, jnp.minimum(nch, 1), slow_chunk, init)

    @pl.when(npairs > 0)
    def _():
        start_dma(1, xbuf0, bbuf0, semx0, semb0)

    def pair_body(p, c):
        c0 = 1 + 2 * p
        wait_dma(c0, xbuf0, bbuf0, semx0, semb0)
        start_dma(c0 + 1, xbuf1, bbuf1, semx1, semb1)
        c = fast_rows(xbuf0, bbuf0, c)

        @pl.when(c0 + 2 < nf)
        def _():
            start_dma(c0 + 2, xbuf0, bbuf0, semx0, semb0)

        wait_dma(c0 + 1, xbuf1, bbuf1, semx1, semb1)
        return fast_rows(xbuf1, bbuf1, c)

    carry = lax.fori_loop(0, npairs, pair_body, carry)
    carry = lax.fori_loop(nf, nch, slow_chunk, carry)

    prev_f, cnt_f, ms_f = carry

    @pl.when(cnt_f > 0.0)
    def _():
        flush(prev_f, cnt_f, ms_f)

    loD = pl.multiple_of(lo * D, 8)
    lo8 = pl.multiple_of(lo, 8)
    pltpu.sync_copy(sums, add_hbm.at[pl.ds(loD, SPS * D)])
    pltpu.sync_copy(maxs, max_hbm.at[pl.ds(loD, SPS * D)])
    pltpu.sync_copy(counts.at[pl.ds(0, SPS)], cnt_hbm.at[pl.ds(lo8, SPS)])


def _mlp_body(cnt_ref, add_ref, max_ref, w1m, w1a, w1x, b1_ref,
              w2_ref, b2_ref, out_ref):
    inv = 1.0 / jnp.maximum(cnt_ref[...], 1.0)
    mean = add_ref[...] * inv
    h = (jnp.dot(mean, w1m[...], preferred_element_type=jnp.float32)
         + jnp.dot(add_ref[...], w1a[...], preferred_element_type=jnp.float32)
         + jnp.dot(max_ref[...], w1x[...], preferred_element_type=jnp.float32)
         + b1_ref[...])
    h = jnp.maximum(h, 0.0)
    out_ref[...] = (jnp.dot(h, w2_ref[...], preferred_element_type=jnp.float32)
                    + b2_ref[...])


BM = SPAD // 4


def kernel(x, batch, W1, b1, W2, b2):
    seg_bounds = jnp.arange(33, dtype=jnp.int32) * SPS
    row_start = jnp.searchsorted(batch, seg_bounds).astype(jnp.int32)
    rs8 = (row_start // 8) * 8
    starts = jnp.concatenate(
        [rs8, jnp.full((15,), N, dtype=jnp.int32)])

    mesh = plsc.VectorSubcoreMesh(core_axis_name="c", subcore_axis_name="s")
    pool = pl.kernel(
        _pool_body,
        out_type=[jax.ShapeDtypeStruct((SPAD * D,), jnp.float32),
                  jax.ShapeDtypeStruct((SPAD * D,), jnp.float32),
                  jax.ShapeDtypeStruct((SPAD,), jnp.float32)],
        mesh=mesh,
        scratch_types=[
            pltpu.VMEM((48,), jnp.int32),
            pltpu.VMEM((CHUNK,), jnp.int32),
            pltpu.VMEM((CHUNK,), jnp.int32),
            pltpu.VMEM((CHUNK * D,), jnp.float32),
            pltpu.VMEM((CHUNK * D,), jnp.float32),
            pltpu.VMEM((SPS * D,), jnp.float32),
            pltpu.VMEM((SPS * D,), jnp.float32),
            pltpu.VMEM((CPAD,), jnp.float32),
            pltpu.SemaphoreType.DMA,
            pltpu.SemaphoreType.DMA,
            pltpu.SemaphoreType.DMA,
            pltpu.SemaphoreType.DMA,
        ],
    )
    add_p, max_p, cnt_p = pool(x.reshape(N * D), batch, starts)
    add_p = add_p.reshape(SPAD, D)
    max_p = max_p.reshape(SPAD, D)
    cnt_p = cnt_p.reshape(SPAD, 1)

    out = pl.pallas_call(
        _mlp_body,
        grid=(SPAD // BM,),
        in_specs=[
            pl.BlockSpec((BM, 1), lambda i: (i, 0)),
            pl.BlockSpec((BM, D), lambda i: (i, 0)),
            pl.BlockSpec((BM, D), lambda i: (i, 0)),
            pl.BlockSpec((D, H), lambda i: (0, 0)),
            pl.BlockSpec((D, H), lambda i: (0, 0)),
            pl.BlockSpec((D, H), lambda i: (0, 0)),
            pl.BlockSpec((1, H), lambda i: (0, 0)),
            pl.BlockSpec((H, O), lambda i: (0, 0)),
            pl.BlockSpec((1, O), lambda i: (0, 0)),
        ],
        out_specs=pl.BlockSpec((BM, O), lambda i: (i, 0)),
        out_shape=jax.ShapeDtypeStruct((SPAD, O), jnp.float32),
    )(cnt_p, add_p, max_p,
      W1[0:D], W1[D:2 * D], W1[2 * D:3 * D],
      b1.reshape(1, H), W2, b2.reshape(1, O))
    return out[:S]


# speedup vs baseline: 10.6696x; 1.2963x over previous
import jax
import jax.numpy as jnp
from jax import lax
from jax.experimental import pallas as pl
from jax.experimental.pallas import tpu as pltpu
from jax.experimental.pallas import tpu_sc as plsc

N = 320000
D = 128
S = 10000
H = 512
O = 128

NW = 32
SPS = 320
SPAD = NW * SPS
CHUNK = 160
NV = D // 16
CPAD = SPS + 16
NEG = -3.0e38


def _pool_body(x_hbm, batch_hbm, starts_hbm,
               add_hbm, max_hbm, cnt_hbm,
               starts_v, bbuf0, bbuf1, xbuf0, xbuf1,
               sums, maxs, counts, blog,
               semx0, semb0, semx1, semb1):
    cid = lax.axis_index("c")
    sid = lax.axis_index("s")
    wid = sid * 2 + cid
    lo = wid * SPS

    pltpu.sync_copy(starts_hbm, starts_v)
    rs8 = starts_v[pl.ds(wid, 16)][0]
    nxt = starts_v[pl.ds(wid + 1, 16)][0]
    re_eff = jnp.minimum(nxt + 8, N)

    zf = jnp.zeros((16,), jnp.float32)

    def zero_seg(i, _):
        for j in range(NV):
            sums[pl.ds(i * D + 16 * j, 16)] = zf
            maxs[pl.ds(i * D + 16 * j, 16)] = zf
        return 0

    lax.fori_loop(0, SPS, zero_seg, 0)

    def zero_cnt(i, _):
        counts[pl.ds(i * 16, 16)] = zf
        return 0

    lax.fori_loop(0, CPAD // 16, zero_cnt, 0)

    nch = (re_eff - rs8 + CHUNK - 1) // CHUNK
    nf_raw = (nxt - rs8) // CHUNK
    nf = jnp.maximum(1 + 2 * ((nf_raw - 1) // 2), 1)
    npairs = (nf - 1) // 2

    def flush2(prev, cnt, ss, ms):
        moff = prev * D
        for j in range(NV):
            sums[pl.ds(moff + 16 * j, 16)] = ss[j]
            maxs[pl.ds(moff + 16 * j, 16)] = ms[j]
        counts[pl.ds(prev, 16)] = jnp.full((16,), cnt, jnp.float32)

    zvs_t = tuple(jnp.zeros((16,), jnp.float32) for _ in range(NV))
    nvs_t = tuple(jnp.full((16,), NEG, jnp.float32) for _ in range(NV))

    def fast_rows(xb, bb, carry):
        prev, cnt, ss, ms = carry

        def load_row(r):
            return tuple(xb[pl.ds(r * D + 16 * j, 16)]
                         for j in range(NV))

        def accum(r, st):
            c, s_, m_ = st
            vs = load_row(r)
            return (c + 1.0,
                    tuple(s_[j] + vs[j] for j in range(NV)),
                    tuple(jnp.maximum(m_[j], vs[j]) for j in range(NV)))

        def pre_group(g, st):
            nb, pb = st
            bv = bb[pl.ds(16 * g, 16)]
            for l in range(16):
                b = bv[l]
                ch = b != pb

                @pl.when(ch)
                def _(nb=nb, g=g, l=l):
                    blog[nb] = 16 * g + l

                nb = nb + jnp.where(ch, 1, 0)
                pb = b
            return (nb, pb)

        nb, _ = lax.fori_loop(0, CHUNK // 16, pre_group,
                              (jnp.int32(0), prev))

        e0 = jnp.where(nb > 0, blog[0], CHUNK)
        cnt1, ss1, ms1 = lax.fori_loop(0, e0, accum, (cnt, ss, ms))

        @pl.when((nb > 0) & (cnt1 > 0.0))
        def _():
            flush2(prev - lo, cnt1, ss1, ms1)

        def run_body(k, _):
            a = blog[k]
            a1 = blog[k + 1]
            rid = bb[pl.ds(a, 16)][0]
            ck, sk, mk = lax.fori_loop(a, a1, accum,
                                       (jnp.float32(0.0), zvs_t, nvs_t))
            flush2(rid - lo, ck, sk, mk)
            return 0

        lax.fori_loop(0, jnp.maximum(nb - 1, 0), run_body, 0)

        af = jnp.where(nb > 0, blog[jnp.maximum(nb - 1, 0)],
                       jnp.int32(CHUNK))
        rf = bb[pl.ds(af, 16)][0]
        cntf, ssf, msf = lax.fori_loop(af, CHUNK, accum,
                                       (jnp.float32(0.0), zvs_t, nvs_t))
        prev2 = jnp.where(nb > 0, rf, prev)
        cnt2 = jnp.where(nb > 0, cntf, cnt1)
        ss2 = tuple(jnp.where(nb > 0, ssf[j], ss1[j]) for j in range(NV))
        ms2 = tuple(jnp.where(nb > 0, msf[j], ms1[j]) for j in range(NV))
        return (prev2, cnt2, ss2, ms2)

    def slow_chunk(ci, carry):
        lstart = rs8 + ci * CHUNK
        astart = pl.multiple_of(jnp.minimum(lstart, N - CHUNK), 8)
        delta = lstart - astart
        pltpu.sync_copy(batch_hbm.at[pl.ds(astart, CHUNK)],
                        bbuf0.at[pl.ds(0, CHUNK)])
        xoff = pl.multiple_of(astart * D, 8)
        pltpu.sync_copy(x_hbm.at[pl.ds(xoff, CHUNK * D)], xbuf0)

        def group_body(g, c):
            bvec = bbuf0[pl.ds(16 * g, 16)]
            for l in range(16):
                prev, cnt, ss, ms = c
                r = 16 * g + l
                rel = bvec[l] - lo
                row_g = astart + r
                valid = ((r >= delta) & (row_g < re_eff)
                         & (rel >= 0) & (rel < SPS))
                vs = tuple(xbuf0[pl.ds(r * D + 16 * j, 16)]
                           for j in range(NV))
                changed = valid & (bvec[l] != prev)

                @pl.when(changed & (cnt > 0.0))
                def _(prev=prev, cnt=cnt, ss=ss, ms=ms):
                    flush2(prev - lo, cnt, ss, ms)

                ss2 = tuple(
                    jnp.where(changed, vs[j],
                              ss[j] + jnp.where(valid, vs[j], 0.0))
                    for j in range(NV))
                ms2 = tuple(
                    jnp.where(changed, vs[j],
                              jnp.maximum(ms[j],
                                          jnp.where(valid, vs[j], NEG)))
                    for j in range(NV))
                cnt2 = jnp.where(changed, 1.0,
                                 jnp.where(valid, cnt + 1.0, cnt))
                prev2 = jnp.where(changed, bvec[l], prev)
                c = (prev2, cnt2, ss2, ms2)
            return c

        return lax.fori_loop(0, CHUNK // 16, group_body, carry)

    def start_dma(ci, xb, bb, sx, sb):
        a8 = pl.multiple_of(rs8 + ci * CHUNK, 8)
        pltpu.async_copy(batch_hbm.at[pl.ds(a8, CHUNK)],
                         bb.at[pl.ds(0, CHUNK)], sb)
        xo = pl.multiple_of(a8 * D, 8)
        pltpu.async_copy(x_hbm.at[pl.ds(xo, CHUNK * D)], xb, sx)

    def wait_dma(ci, xb, bb, sx, sb):
        a8 = pl.multiple_of(rs8 + ci * CHUNK, 8)
        pltpu.make_async_copy(batch_hbm.at[pl.ds(a8, CHUNK)],
                              bb.at[pl.ds(0, CHUNK)], sb).wait()
        xo = pl.multiple_of(a8 * D, 8)
        pltpu.make_async_copy(x_hbm.at[pl.ds(xo, CHUNK * D)], xb,
                              sx).wait()

    zvs = tuple(zf for _ in range(NV))
    init = (jnp.int32(-1), jnp.float32(0.0), zvs, zvs)

    carry = lax.fori_loop(0, jnp.minimum(nch, 1), slow_chunk, init)

    @pl.when(npairs > 0)
    def _():
        start_dma(1, xbuf0, bbuf0, semx0, semb0)

    def pair_body(p, c):
        c0 = 1 + 2 * p
        wait_dma(c0, xbuf0, bbuf0, semx0, semb0)
        start_dma(c0 + 1, xbuf1, bbuf1, semx1, semb1)
        c = fast_rows(xbuf0, bbuf0, c)

        @pl.when(c0 + 2 < nf)
        def _():
            start_dma(c0 + 2, xbuf0, bbuf0, semx0, semb0)

        wait_dma(c0 + 1, xbuf1, bbuf1, semx1, semb1)
        return fast_rows(xbuf1, bbuf1, c)

    carry = lax.fori_loop(0, npairs, pair_body, carry)
    carry = lax.fori_loop(nf, nch, slow_chunk, carry)

    prev_f, cnt_f, ss_f, ms_f = carry

    @pl.when(cnt_f > 0.0)
    def _():
        flush2(prev_f - lo, cnt_f, ss_f, ms_f)

    loD = pl.multiple_of(lo * D, 8)
    lo8 = pl.multiple_of(lo, 8)
    pltpu.sync_copy(sums, add_hbm.at[pl.ds(loD, SPS * D)])
    pltpu.sync_copy(maxs, max_hbm.at[pl.ds(loD, SPS * D)])
    pltpu.sync_copy(counts.at[pl.ds(0, SPS)], cnt_hbm.at[pl.ds(lo8, SPS)])


def _mlp_body(cnt_ref, add_ref, max_ref, w1m, w1a, w1x, b1_ref,
              w2_ref, b2_ref, out_ref):
    inv = 1.0 / jnp.maximum(cnt_ref[...], 1.0)
    mean = add_ref[...] * inv
    h = (jnp.dot(mean, w1m[...], preferred_element_type=jnp.float32)
         + jnp.dot(add_ref[...], w1a[...], preferred_element_type=jnp.float32)
         + jnp.dot(max_ref[...], w1x[...], preferred_element_type=jnp.float32)
         + b1_ref[...])
    h = jnp.maximum(h, 0.0)
    out_ref[...] = (jnp.dot(h, w2_ref[...], preferred_element_type=jnp.float32)
                    + b2_ref[...])


BM = SPAD // 4


def kernel(x, batch, W1, b1, W2, b2):
    seg_bounds = jnp.arange(33, dtype=jnp.int32) * SPS
    row_start = jnp.searchsorted(batch, seg_bounds).astype(jnp.int32)
    rs8 = (row_start // 8) * 8
    starts = jnp.concatenate(
        [rs8, jnp.full((15,), N, dtype=jnp.int32)])

    mesh = plsc.VectorSubcoreMesh(core_axis_name="c", subcore_axis_name="s")
    pool = pl.kernel(
        _pool_body,
        out_type=[jax.ShapeDtypeStruct((SPAD * D,), jnp.float32),
                  jax.ShapeDtypeStruct((SPAD * D,), jnp.float32),
                  jax.ShapeDtypeStruct((SPAD,), jnp.float32)],
        mesh=mesh,
        scratch_types=[
            pltpu.VMEM((48,), jnp.int32),
            pltpu.VMEM((CHUNK + 16,), jnp.int32),
            pltpu.VMEM((CHUNK + 16,), jnp.int32),
            pltpu.VMEM((CHUNK * D,), jnp.float32),
            pltpu.VMEM((CHUNK * D,), jnp.float32),
            pltpu.VMEM((SPS * D,), jnp.float32),
            pltpu.VMEM((SPS * D,), jnp.float32),
            pltpu.VMEM((CPAD,), jnp.float32),
            pltpu.SMEM((CHUNK + 16,), jnp.int32),
            pltpu.SemaphoreType.DMA,
            pltpu.SemaphoreType.DMA,
            pltpu.SemaphoreType.DMA,
            pltpu.SemaphoreType.DMA,
        ],
    )
    add_p, max_p, cnt_p = pool(x.reshape(N * D), batch, starts)
    add_p = add_p.reshape(SPAD, D)
    max_p = max_p.reshape(SPAD, D)
    cnt_p = cnt_p.reshape(SPAD, 1)

    out = pl.pallas_call(
        _mlp_body,
        grid=(SPAD // BM,),
        in_specs=[
            pl.BlockSpec((BM, 1), lambda i: (i, 0)),
            pl.BlockSpec((BM, D), lambda i: (i, 0)),
            pl.BlockSpec((BM, D), lambda i: (i, 0)),
            pl.BlockSpec((D, H), lambda i: (0, 0)),
            pl.BlockSpec((D, H), lambda i: (0, 0)),
            pl.BlockSpec((D, H), lambda i: (0, 0)),
            pl.BlockSpec((1, H), lambda i: (0, 0)),
            pl.BlockSpec((H, O), lambda i: (0, 0)),
            pl.BlockSpec((1, O), lambda i: (0, 0)),
        ],
        out_specs=pl.BlockSpec((BM, O), lambda i: (i, 0)),
        out_shape=jax.ShapeDtypeStruct((SPAD, O), jnp.float32),
    )(cnt_p, add_p, max_p,
      W1[0:D], W1[D:2 * D], W1[2 * D:3 * D],
      b1.reshape(1, H), W2, b2.reshape(1, O))
    return out[:S]


# speedup vs baseline: 13.3704x; 1.2531x over previous
import jax
import jax.numpy as jnp
from jax import lax
from jax.experimental import pallas as pl
from jax.experimental.pallas import tpu as pltpu
from jax.experimental.pallas import tpu_sc as plsc

N = 320000
D = 128
S = 10000
H = 512
O = 128

NW = 32
SPS = 320
SPAD = NW * SPS
CHUNK = 160
NV = D // 16
CPAD = SPS + 16
NEG = -3.0e38


def _pool_body(x_hbm, batch_hbm, starts_hbm,
               add_hbm, max_hbm, cnt_hbm,
               starts_v, bbuf0, bbuf1, xbuf0, xbuf1,
               sums, maxs, counts, blog,
               semx0, semb0, semx1, semb1):
    cid = lax.axis_index("c")
    sid = lax.axis_index("s")
    wid = sid * 2 + cid
    lo = wid * SPS

    pltpu.sync_copy(starts_hbm, starts_v)
    rs8 = starts_v[pl.ds(wid, 16)][0]
    nxt = starts_v[pl.ds(wid + 1, 16)][0]
    re_eff = jnp.minimum(nxt + 72, N)

    zf = jnp.zeros((16,), jnp.float32)

    def zero_seg(i, _):
        for j in range(NV):
            sums[pl.ds(i * D + 16 * j, 16)] = zf
            maxs[pl.ds(i * D + 16 * j, 16)] = zf
        return 0

    lax.fori_loop(0, SPS, zero_seg, 0)

    def zero_cnt(i, _):
        counts[pl.ds(i * 16, 16)] = zf
        return 0

    lax.fori_loop(0, CPAD // 16, zero_cnt, 0)

    nch = (re_eff - rs8 + CHUNK - 1) // CHUNK
    nf_raw = (nxt - rs8) // CHUNK
    nf = jnp.maximum(1 + 2 * ((nf_raw - 1) // 2), 1)
    npairs = (nf - 1) // 2

    def flush2(prev, cnt, ss, ms):
        moff = prev * D
        for j in range(NV):
            sums[pl.ds(moff + 16 * j, 16)] = ss[j]
            maxs[pl.ds(moff + 16 * j, 16)] = ms[j]
        counts[pl.ds(prev, 16)] = jnp.full((16,), cnt, jnp.float32)

    zvs_t = tuple(jnp.zeros((16,), jnp.float32) for _ in range(NV))
    nvs_t = tuple(jnp.full((16,), NEG, jnp.float32) for _ in range(NV))

    def fast_rows(xb, bb, carry):
        prev, cnt, ss, ms = carry

        def load_row(r):
            return tuple(xb[pl.ds(r * D + 16 * j, 16)]
                         for j in range(NV))

        def accum(r, st):
            c, s_, m_ = st
            vs = load_row(r)
            return (c + 1.0,
                    tuple(s_[j] + vs[j] for j in range(NV)),
                    tuple(jnp.maximum(m_[j], vs[j]) for j in range(NV)))

        def pre_group(g, st):
            nb, pb = st
            bv = bb[pl.ds(16 * g, 16)]
            for l in range(16):
                b = bv[l]
                ch = b != pb

                @pl.when(ch)
                def _(nb=nb, g=g, l=l):
                    blog[nb] = 16 * g + l

                nb = nb + jnp.where(ch, 1, 0)
                pb = b
            return (nb, pb)

        nb, _ = lax.fori_loop(0, CHUNK // 16, pre_group,
                              (jnp.int32(0), prev))

        e0 = jnp.where(nb > 0, blog[0], CHUNK)
        cnt1, ss1, ms1 = lax.fori_loop(0, e0, accum, (cnt, ss, ms))

        @pl.when((nb > 0) & (cnt1 > 0.0))
        def _():
            flush2(prev - lo, cnt1, ss1, ms1)

        def run_body(k, _):
            a = blog[k]
            a1 = blog[k + 1]
            rid = bb[pl.ds(a, 16)][0]
            ck, sk, mk = lax.fori_loop(a, a1, accum,
                                       (jnp.float32(0.0), zvs_t, nvs_t))
            flush2(rid - lo, ck, sk, mk)
            return 0

        lax.fori_loop(0, jnp.maximum(nb - 1, 0), run_body, 0)

        af = jnp.where(nb > 0, blog[jnp.maximum(nb - 1, 0)],
                       jnp.int32(CHUNK))
        rf = bb[pl.ds(af, 16)][0]
        cntf, ssf, msf = lax.fori_loop(af, CHUNK, accum,
                                       (jnp.float32(0.0), zvs_t, nvs_t))
        prev2 = jnp.where(nb > 0, rf, prev)
        cnt2 = jnp.where(nb > 0, cntf, cnt1)
        ss2 = tuple(jnp.where(nb > 0, ssf[j], ss1[j]) for j in range(NV))
        ms2 = tuple(jnp.where(nb > 0, msf[j], ms1[j]) for j in range(NV))
        return (prev2, cnt2, ss2, ms2)

    def slow_chunk(ci, carry):
        lstart = rs8 + ci * CHUNK
        astart = pl.multiple_of(jnp.minimum(lstart, N - CHUNK), 8)
        delta = lstart - astart
        pltpu.sync_copy(batch_hbm.at[pl.ds(astart, CHUNK)],
                        bbuf0.at[pl.ds(0, CHUNK)])
        xoff = pl.multiple_of(astart * D, 8)
        pltpu.sync_copy(x_hbm.at[pl.ds(xoff, CHUNK * D)], xbuf0)

        def group_body(g, c):
            bvec = bbuf0[pl.ds(16 * g, 16)]
            for l in range(16):
                prev, cnt, ss, ms = c
                r = 16 * g + l
                rel = bvec[l] - lo
                row_g = astart + r
                valid = ((r >= delta) & (row_g < re_eff)
                         & (rel >= 0) & (rel < SPS))
                vs = tuple(xbuf0[pl.ds(r * D + 16 * j, 16)]
                           for j in range(NV))
                changed = valid & (bvec[l] != prev)

                @pl.when(changed & (cnt > 0.0))
                def _(prev=prev, cnt=cnt, ss=ss, ms=ms):
                    flush2(prev - lo, cnt, ss, ms)

                ss2 = tuple(
                    jnp.where(changed, vs[j],
                              ss[j] + jnp.where(valid, vs[j], 0.0))
                    for j in range(NV))
                ms2 = tuple(
                    jnp.where(changed, vs[j],
                              jnp.maximum(ms[j],
                                          jnp.where(valid, vs[j], NEG)))
                    for j in range(NV))
                cnt2 = jnp.where(changed, 1.0,
                                 jnp.where(valid, cnt + 1.0, cnt))
                prev2 = jnp.where(changed, bvec[l], prev)
                c = (prev2, cnt2, ss2, ms2)
            return c

        return lax.fori_loop(0, CHUNK // 16, group_body, carry)

    def start_dma(ci, xb, bb, sx, sb):
        a8 = pl.multiple_of(rs8 + ci * CHUNK, 8)
        pltpu.async_copy(batch_hbm.at[pl.ds(a8, CHUNK)],
                         bb.at[pl.ds(0, CHUNK)], sb)
        xo = pl.multiple_of(a8 * D, 8)
        pltpu.async_copy(x_hbm.at[pl.ds(xo, CHUNK * D)], xb, sx)

    def wait_dma(ci, xb, bb, sx, sb):
        a8 = pl.multiple_of(rs8 + ci * CHUNK, 8)
        pltpu.make_async_copy(batch_hbm.at[pl.ds(a8, CHUNK)],
                              bb.at[pl.ds(0, CHUNK)], sb).wait()
        xo = pl.multiple_of(a8 * D, 8)
        pltpu.make_async_copy(x_hbm.at[pl.ds(xo, CHUNK * D)], xb,
                              sx).wait()

    zvs = tuple(zf for _ in range(NV))
    init = (jnp.int32(-1), jnp.float32(0.0), zvs, zvs)

    carry = lax.fori_loop(0, jnp.minimum(nch, 1), slow_chunk, init)

    @pl.when(npairs > 0)
    def _():
        start_dma(1, xbuf0, bbuf0, semx0, semb0)

    def pair_body(p, c):
        c0 = 1 + 2 * p
        wait_dma(c0, xbuf0, bbuf0, semx0, semb0)
        start_dma(c0 + 1, xbuf1, bbuf1, semx1, semb1)
        c = fast_rows(xbuf0, bbuf0, c)

        @pl.when(c0 + 2 < nf)
        def _():
            start_dma(c0 + 2, xbuf0, bbuf0, semx0, semb0)

        wait_dma(c0 + 1, xbuf1, bbuf1, semx1, semb1)
        return fast_rows(xbuf1, bbuf1, c)

    carry = lax.fori_loop(0, npairs, pair_body, carry)
    carry = lax.fori_loop(nf, nch, slow_chunk, carry)

    prev_f, cnt_f, ss_f, ms_f = carry

    @pl.when(cnt_f > 0.0)
    def _():
        flush2(prev_f - lo, cnt_f, ss_f, ms_f)

    loD = pl.multiple_of(lo * D, 8)
    lo8 = pl.multiple_of(lo, 8)
    pltpu.sync_copy(sums, add_hbm.at[pl.ds(loD, SPS * D)])
    pltpu.sync_copy(maxs, max_hbm.at[pl.ds(loD, SPS * D)])
    pltpu.sync_copy(counts.at[pl.ds(0, SPS)], cnt_hbm.at[pl.ds(lo8, SPS)])


def _mlp_body(cnt_ref, add_ref, max_ref, w1m, w1a, w1x, b1_ref,
              w2_ref, b2_ref, out_ref):
    inv = 1.0 / jnp.maximum(cnt_ref[...], 1.0)
    mean = add_ref[...] * inv
    h = (jnp.dot(mean, w1m[...], preferred_element_type=jnp.float32)
         + jnp.dot(add_ref[...], w1a[...], preferred_element_type=jnp.float32)
         + jnp.dot(max_ref[...], w1x[...], preferred_element_type=jnp.float32)
         + b1_ref[...])
    h = jnp.maximum(h, 0.0)
    out_ref[...] = (jnp.dot(h, w2_ref[...], preferred_element_type=jnp.float32)
                    + b2_ref[...])


BM = SPAD // 4


def kernel(x, batch, W1, b1, W2, b2):
    seg_bounds = jnp.arange(33, dtype=jnp.int32) * SPS
    sub = batch[::64]
    p = jnp.sum(sub[None, :] < seg_bounds[:, None],
                axis=1).astype(jnp.int32)
    rs8 = jnp.maximum(p * 64 - 64, 0)
    starts = jnp.concatenate(
        [rs8, jnp.full((15,), N, dtype=jnp.int32)])

    mesh = plsc.VectorSubcoreMesh(core_axis_name="c", subcore_axis_name="s")
    pool = pl.kernel(
        _pool_body,
        out_type=[jax.ShapeDtypeStruct((SPAD * D,), jnp.float32),
                  jax.ShapeDtypeStruct((SPAD * D,), jnp.float32),
                  jax.ShapeDtypeStruct((SPAD,), jnp.float32)],
        mesh=mesh,
        scratch_types=[
            pltpu.VMEM((48,), jnp.int32),
            pltpu.VMEM((CHUNK + 16,), jnp.int32),
            pltpu.VMEM((CHUNK + 16,), jnp.int32),
            pltpu.VMEM((CHUNK * D,), jnp.float32),
            pltpu.VMEM((CHUNK * D,), jnp.float32),
            pltpu.VMEM((SPS * D,), jnp.float32),
            pltpu.VMEM((SPS * D,), jnp.float32),
            pltpu.VMEM((CPAD,), jnp.float32),
            pltpu.SMEM((CHUNK + 16,), jnp.int32),
            pltpu.SemaphoreType.DMA,
            pltpu.SemaphoreType.DMA,
            pltpu.SemaphoreType.DMA,
            pltpu.SemaphoreType.DMA,
        ],
    )
    add_p, max_p, cnt_p = pool(x.reshape(N * D), batch, starts)
    add_p = add_p.reshape(SPAD, D)
    max_p = max_p.reshape(SPAD, D)
    cnt_p = cnt_p.reshape(SPAD, 1)

    out = pl.pallas_call(
        _mlp_body,
        grid=(SPAD // BM,),
        in_specs=[
            pl.BlockSpec((BM, 1), lambda i: (i, 0)),
            pl.BlockSpec((BM, D), lambda i: (i, 0)),
            pl.BlockSpec((BM, D), lambda i: (i, 0)),
            pl.BlockSpec((D, H), lambda i: (0, 0)),
            pl.BlockSpec((D, H), lambda i: (0, 0)),
            pl.BlockSpec((D, H), lambda i: (0, 0)),
            pl.BlockSpec((1, H), lambda i: (0, 0)),
            pl.BlockSpec((H, O), lambda i: (0, 0)),
            pl.BlockSpec((1, O), lambda i: (0, 0)),
        ],
        out_specs=pl.BlockSpec((BM, O), lambda i: (i, 0)),
        out_shape=jax.ShapeDtypeStruct((SPAD, O), jnp.float32),
    )(cnt_p, add_p, max_p,
      W1[0:D], W1[D:2 * D], W1[2 * D:3 * D],
      b1.reshape(1, H), W2, b2.reshape(1, O))
    return out[:S]
